# Initial kernel scaffold; baseline (speedup 1.0000x reference)
import jax
import jax.numpy as jnp
from jax.experimental import pallas as pl


def kernel(x, edge_index, params):
    raise NotImplementedError("write your pallas kernel here")


# speedup vs baseline: 2.1822x; 2.1822x over previous
import functools

import jax
import jax.numpy as jnp
from jax import lax
from jax.experimental import pallas as pl
from jax.experimental.pallas import tpu as pltpu
from jax.experimental.pallas import tpu_sc as plsc

N = 10000
D = 128
E = 320000

_NC = 2
_NS = 16
_NW = _NC * _NS
_EPB = 128
_EPAD = 327680
_TPW = _EPAD // (_NW * _EPB)
_ACC_ROWS = N + 8
_ZROWS = 8
_RPT = 624

def _seg_sum_body(h_hbm, src_hbm, dst_hbm, out_hbm,
                  src_i, dst_i, rows_v, zbuf, acc, sem):
    cid = lax.axis_index("c")
    sid = lax.axis_index("s")
    wid = sid * _NC + cid

    def _z(i, _):
        zbuf[i // 8, pl.ds((i % 8) * 16, 16)] = jnp.zeros((16,), jnp.float32)
        return 0
    lax.fori_loop(0, _ZROWS * (D // 16), _z, 0)

    def _zc(i, _):
        pltpu.sync_copy(zbuf, acc.at[pl.ds(sid * _RPT + i * _ZROWS, _ZROWS)])
        return 0
    lax.fori_loop(0, _RPT // _ZROWS, _zc, 0)

    @pl.when(sid == _NS - 1)
    def _zc_tail():
        def _zt(i, _):
            pltpu.sync_copy(zbuf, acc.at[pl.ds(_NS * _RPT + i * _ZROWS, _ZROWS)])
            return 0
        lax.fori_loop(0, (_ACC_ROWS - _NS * _RPT) // _ZROWS, _zt, 0)

    plsc.subcore_barrier()

    pltpu.sync_copy(src_hbm.at[pl.ds(wid * _TPW, _TPW)], src_i)
    pltpu.sync_copy(dst_hbm.at[pl.ds(wid * _TPW, _TPW)], dst_i)

    def _et(t, _):
        pltpu.async_copy(h_hbm.at[src_i.at[t]], rows_v, sem).wait()
        pltpu.sync_copy(rows_v, acc.at[dst_i.at[t]], add=True)
        return 0
    lax.fori_loop(0, _TPW, _et, 0)

    plsc.subcore_barrier()

    @pl.when(sid < _NS - 1)
    def _dump():
        pltpu.sync_copy(acc.at[pl.ds(sid * _RPT, _RPT)],
                        out_hbm.at[cid, pl.ds(sid * _RPT, _RPT)])

    @pl.when(sid == _NS - 1)
    def _dump_tail():
        pltpu.sync_copy(acc.at[pl.ds((_NS - 1) * _RPT, N - (_NS - 1) * _RPT)],
                        out_hbm.at[cid, pl.ds((_NS - 1) * _RPT,
                                              N - (_NS - 1) * _RPT)])


@functools.cache
def _get_seg_sum():
    mesh = plsc.VectorSubcoreMesh(core_axis_name="c", subcore_axis_name="s")
    return pl.kernel(
        _seg_sum_body,
        out_type=jax.ShapeDtypeStruct((_NC, N, D), jnp.float32),
        mesh=mesh,
        scratch_types=[
            pltpu.VMEM((_TPW, _EPB), jnp.int32),
            pltpu.VMEM((_TPW, _EPB), jnp.int32),
            pltpu.VMEM((_EPB, D), jnp.float32),
            pltpu.VMEM((_ZROWS, D), jnp.float32),
            pltpu.VMEM_SHARED((_ACC_ROWS, D), jnp.float32),
            pltpu.SemaphoreType.DMA,
        ],
    )


def _gin_layer_body(h_ref, agg_ref, eps_ref, w1_ref, b1_ref, w2_ref, b2_ref,
                    g_ref, be_ref, o_ref):
    eps = eps_ref[0, 0]
    ht = (1.0 + eps) * h_ref[...] + agg_ref[0] + agg_ref[1]
    z = jnp.dot(ht, w1_ref[...], preferred_element_type=jnp.float32) + b1_ref[...]
    z = jnp.maximum(z, 0.0)
    z = jnp.dot(z, w2_ref[...], preferred_element_type=jnp.float32) + b2_ref[...]
    m = jnp.mean(z, axis=0, keepdims=True)
    v = jnp.mean((z - m) ** 2, axis=0, keepdims=True)
    zn = (z - m) * lax.rsqrt(v + 1e-5) * g_ref[...] + be_ref[...]
    o_ref[...] = jnp.maximum(zn, 0.0)


_gin_layer = pl.pallas_call(
    _gin_layer_body,
    out_shape=jax.ShapeDtypeStruct((N, D), jnp.float32),
)


def _head_body(h_ref, w0_ref, b0_ref, g0_ref, be0_ref,
               w1_ref, b1_ref, g1_ref, be1_ref, w2_ref, b2_ref, o_ref):
    def bn_relu(z, g, be):
        m = jnp.mean(z, axis=0, keepdims=True)
        v = jnp.mean((z - m) ** 2, axis=0, keepdims=True)
        return jnp.maximum((z - m) * lax.rsqrt(v + 1e-5) * g + be, 0.0)

    h = h_ref[...]
    h = bn_relu(jnp.dot(h, w0_ref[...], preferred_element_type=jnp.float32) + b0_ref[...],
                g0_ref[...], be0_ref[...])
    h = bn_relu(jnp.dot(h, w1_ref[...], preferred_element_type=jnp.float32) + b1_ref[...],
                g1_ref[...], be1_ref[...])
    o_ref[...] = jnp.dot(h, w2_ref[...], preferred_element_type=jnp.float32) + b2_ref[...]


_head = pl.pallas_call(
    _head_body,
    out_shape=jax.ShapeDtypeStruct((N, 3), jnp.float32),
)


def kernel(x, edge_index, params):
    src = edge_index[0]
    dst = edge_index[1]
    perm = jnp.argsort(dst, stable=True)
    src = src[perm]
    dst = dst[perm]
    pad = _EPAD - E
    src2d = jnp.concatenate(
        [src, jnp.zeros((pad,), jnp.int32)]).reshape(_EPAD // _EPB, _EPB)
    dst2d = jnp.concatenate(
        [dst, jnp.full((pad,), N, jnp.int32)]).reshape(_EPAD // _EPB, _EPB)

    seg_sum = _get_seg_sum()
    h = x
    for i in range(3):
        agg = seg_sum(h, src2d, dst2d)
        h = _gin_layer(
            h, agg,
            params['eps%d' % i].reshape(1, 1),
            params['W1_%d' % i], params['b1_%d' % i].reshape(1, D),
            params['W2_%d' % i], params['b2_%d' % i].reshape(1, D),
            params['g%d' % i].reshape(1, D), params['be%d' % i].reshape(1, D),
        )
    return _head(
        h,
        params['Wc0'], params['bc0'].reshape(1, D),
        params['gc0'].reshape(1, D), params['bec0'].reshape(1, D),
        params['Wc1'], params['bc1'].reshape(1, D),
        params['gc1'].reshape(1, D), params['bec1'].reshape(1, D),
        params['Wc2'], params['bc2'].reshape(1, 3),
    )


# speedup vs baseline: 2.3603x; 1.0816x over previous
import functools

import jax
import jax.numpy as jnp
from jax import lax
from jax.experimental import pallas as pl
from jax.experimental.pallas import tpu as pltpu
from jax.experimental.pallas import tpu_sc as plsc

N = 10000
D = 128
E = 320000

_NC = 2
_NS = 16
_NW = _NC * _NS
_EPB = 128
_EPAD = 327680
_TPW = _EPAD // (_NW * _EPB)
_ACC_ROWS = N + 8
_ZROWS = 8
_RPT = 624
_NBUF = 2
_HALF = _TPW // 2

def _seg_sum_body(h_hbm, src_hbm, dst_hbm, out_hbm,
                  src_i, dst_i, rows_v, zbuf, acc, sem0, sem1):
    cid = lax.axis_index("c")
    sid = lax.axis_index("s")
    wid = sid * _NC + cid

    def _z(i, _):
        zbuf[i // 8, pl.ds((i % 8) * 16, 16)] = jnp.zeros((16,), jnp.float32)
        return 0
    lax.fori_loop(0, _ZROWS * (D // 16), _z, 0)

    def _zc(i, _):
        pltpu.sync_copy(zbuf, acc.at[pl.ds(sid * _RPT + i * _ZROWS, _ZROWS)])
        return 0
    lax.fori_loop(0, _RPT // _ZROWS, _zc, 0)

    @pl.when(sid == _NS - 1)
    def _zc_tail():
        def _zt(i, _):
            pltpu.sync_copy(zbuf, acc.at[pl.ds(_NS * _RPT + i * _ZROWS, _ZROWS)])
            return 0
        lax.fori_loop(0, (_ACC_ROWS - _NS * _RPT) // _ZROWS, _zt, 0)

    plsc.subcore_barrier()

    sems = (sem0, sem1)
    for ph in range(_TPW // _HALF):
        pltpu.sync_copy(src_hbm.at[pl.ds(wid * _TPW + ph * _HALF, _HALF)],
                        src_i)
        pltpu.sync_copy(dst_hbm.at[pl.ds(wid * _TPW + ph * _HALF, _HALF)],
                        dst_i)
        for j in range(_NBUF):
            pltpu.async_copy(h_hbm.at[src_i.at[j]], rows_v.at[j], sems[j])

        def _et(g, _):
            base = g * _NBUF
            for j in range(_NBUF):
                t = base + j
                pltpu.make_async_copy(
                    h_hbm.at[src_i.at[t]], rows_v.at[j], sems[j]).wait()
                pltpu.sync_copy(rows_v.at[j], acc.at[dst_i.at[t]], add=True)

                @pl.when(t + _NBUF < _HALF)
                def _issue():
                    pltpu.async_copy(
                        h_hbm.at[src_i.at[t + _NBUF]], rows_v.at[j], sems[j])
            return 0
        lax.fori_loop(0, _HALF // _NBUF, _et, 0)

    plsc.subcore_barrier()

    @pl.when(sid < _NS - 1)
    def _dump():
        pltpu.sync_copy(acc.at[pl.ds(sid * _RPT, _RPT)],
                        out_hbm.at[cid, pl.ds(sid * _RPT, _RPT)])

    @pl.when(sid == _NS - 1)
    def _dump_tail():
        pltpu.sync_copy(acc.at[pl.ds((_NS - 1) * _RPT, N - (_NS - 1) * _RPT)],
                        out_hbm.at[cid, pl.ds((_NS - 1) * _RPT,
                                              N - (_NS - 1) * _RPT)])


@functools.cache
def _get_seg_sum():
    mesh = plsc.VectorSubcoreMesh(core_axis_name="c", subcore_axis_name="s")
    return pl.kernel(
        _seg_sum_body,
        out_type=jax.ShapeDtypeStruct((_NC, N, D), jnp.float32),
        mesh=mesh,
        scratch_types=[
            pltpu.VMEM((_HALF, _EPB), jnp.int32),
            pltpu.VMEM((_HALF, _EPB), jnp.int32),
            pltpu.VMEM((_NBUF, _EPB, D), jnp.float32),
            pltpu.VMEM((_ZROWS, D), jnp.float32),
            pltpu.VMEM_SHARED((_ACC_ROWS, D), jnp.float32),
            pltpu.SemaphoreType.DMA,
            pltpu.SemaphoreType.DMA,
        ],
    )


def _gin_layer_body(h_ref, agg_ref, eps_ref, w1_ref, b1_ref, w2_ref, b2_ref,
                    g_ref, be_ref, o_ref):
    eps = eps_ref[0, 0]
    ht = (1.0 + eps) * h_ref[...] + agg_ref[0] + agg_ref[1]
    z = jnp.dot(ht, w1_ref[...], preferred_element_type=jnp.float32) + b1_ref[...]
    z = jnp.maximum(z, 0.0)
    z = jnp.dot(z, w2_ref[...], preferred_element_type=jnp.float32) + b2_ref[...]
    m = jnp.mean(z, axis=0, keepdims=True)
    v = jnp.mean((z - m) ** 2, axis=0, keepdims=True)
    zn = (z - m) * lax.rsqrt(v + 1e-5) * g_ref[...] + be_ref[...]
    o_ref[...] = jnp.maximum(zn, 0.0)


_gin_layer = pl.pallas_call(
    _gin_layer_body,
    out_shape=jax.ShapeDtypeStruct((N, D), jnp.float32),
)


def _head_body(h_ref, w0_ref, b0_ref, g0_ref, be0_ref,
               w1_ref, b1_ref, g1_ref, be1_ref, w2_ref, b2_ref, o_ref):
    def bn_relu(z, g, be):
        m = jnp.mean(z, axis=0, keepdims=True)
        v = jnp.mean((z - m) ** 2, axis=0, keepdims=True)
        return jnp.maximum((z - m) * lax.rsqrt(v + 1e-5) * g + be, 0.0)

    h = h_ref[...]
    h = bn_relu(jnp.dot(h, w0_ref[...], preferred_element_type=jnp.float32) + b0_ref[...],
                g0_ref[...], be0_ref[...])
    h = bn_relu(jnp.dot(h, w1_ref[...], preferred_element_type=jnp.float32) + b1_ref[...],
                g1_ref[...], be1_ref[...])
    o_ref[...] = jnp.dot(h, w2_ref[...], preferred_element_type=jnp.float32) + b2_ref[...]


_head = pl.pallas_call(
    _head_body,
    out_shape=jax.ShapeDtypeStruct((N, 3), jnp.float32),
)


def kernel(x, edge_index, params):
    src = edge_index[0]
    dst = edge_index[1]
    perm = jnp.argsort(dst, stable=True)
    src = src[perm]
    dst = dst[perm]
    pad = _EPAD - E
    src2d = jnp.concatenate(
        [src, jnp.zeros((pad,), jnp.int32)]).reshape(_EPAD // _EPB, _EPB)
    dst2d = jnp.concatenate(
        [dst, jnp.full((pad,), N, jnp.int32)]).reshape(_EPAD // _EPB, _EPB)

    seg_sum = _get_seg_sum()
    h = x
    for i in range(3):
        agg = seg_sum(h, src2d, dst2d)
        h = _gin_layer(
            h, agg,
            params['eps%d' % i].reshape(1, 1),
            params['W1_%d' % i], params['b1_%d' % i].reshape(1, D),
            params['W2_%d' % i], params['b2_%d' % i].reshape(1, D),
            params['g%d' % i].reshape(1, D), params['be%d' % i].reshape(1, D),
        )
    return _head(
        h,
        params['Wc0'], params['bc0'].reshape(1, D),
        params['gc0'].reshape(1, D), params['bec0'].reshape(1, D),
        params['Wc1'], params['bc1'].reshape(1, D),
        params['gc1'].reshape(1, D), params['bec1'].reshape(1, D),
        params['Wc2'], params['bc2'].reshape(1, 3),
    )
